# baseline (speedup 1.0000x reference)
import jax
import jax.numpy as jnp
from jax import lax
from jax.experimental import pallas as pl
from jax.experimental.pallas import tpu as pltpu
from jax.experimental.pallas import tpu_sc as plsc

N = 100000
E = 3200000
NC = 2
NS = 16
L = 16
NW = NC * NS
NPAD = 100352
RPW = NPAD // NW
RPS = NPAD // NS
G = 80
K = 2000
GPC = K // G
EPW = E // NW
NCHUNK = EPW // K
EG = E // G
GPW = EPW // G
NU_MOL = 1.5e-5
EPS = 1e-8


def _edge_body(ei3, attr3, pred, zrows, cconst, acc_out, sm_out,
               rowi, coli, attrb, rowg, colg, chan, smbuf, acc_sh,
               gsem, ssem):
    cid = lax.axis_index("c")
    sid = lax.axis_index("s")
    wid = cid * NS + sid
    pltpu.sync_copy(zrows, acc_sh.at[pl.ds(sid * RPS, RPS)])
    pltpu.sync_copy(cconst, chan)
    plsc.subcore_barrier()

    iota = lax.iota(jnp.int32, L)
    zi = jnp.zeros((L,), jnp.int32)
    oi = jnp.ones((L,), jnp.int32)
    gbase0 = wid * GPW

    def chunk_body(c, smooth):
        gb = gbase0 + c * GPC
        pltpu.sync_copy(ei3.at[0, pl.ds(gb, GPC)], rowi)
        pltpu.sync_copy(ei3.at[1, pl.ds(gb, GPC)], coli)
        pltpu.sync_copy(attr3.at[pl.ds(gb, GPC)], attrb)
        dsc = [pltpu.async_copy(pred.at[rowi.at[g]],
                                rowg.at[pl.ds(g * G, G)], gsem)
               for g in range(GPC)]
        dsc += [pltpu.async_copy(pred.at[coli.at[g]],
                                 colg.at[pl.ds(g * G, G)], gsem)
                for g in range(GPC)]
        for d in dsc:
            d.wait()

        def group_body(g, sm):
            gv = zi + g
            for t in range(G // L):
                e16 = iota + (t * L)
                erow = gv * G + e16
                dx = plsc.load_gather(attrb, [gv, e16, zi])
                dy = plsc.load_gather(attrb, [gv, e16, oi])
                u_r = plsc.load_gather(rowg, [erow, zi])
                v_r = plsc.load_gather(rowg, [erow, oi])
                p_r = plsc.load_gather(rowg, [erow, zi + 2])
                q_r = plsc.load_gather(rowg, [erow, zi + 3])
                u_c = plsc.load_gather(colg, [erow, zi])
                v_c = plsc.load_gather(colg, [erow, oi])
                p_c = plsc.load_gather(colg, [erow, zi + 2])
                q_c = plsc.load_gather(colg, [erow, zi + 3])
                rdx = 1.0 / (dx + EPS)
                rdy = 1.0 / (dy + EPS)
                rdx2 = 1.0 / (dx * dx + EPS)
                rdy2 = 1.0 / (dy * dy + EPS)
                du = u_c - u_r
                dv = v_c - v_r
                dp = p_c - p_r
                dq = q_c - q_r
                plsc.store_scatter(chan, [erow, zi], du * rdx + dv * rdy)
                plsc.store_scatter(chan, [erow, oi], dp * rdx)
                plsc.store_scatter(chan, [erow, zi + 2], dp * rdy)
                plsc.store_scatter(chan, [erow, zi + 3], du * rdx2)
                plsc.store_scatter(chan, [erow, zi + 4], dv * rdy2)
                plsc.store_scatter(chan, [erow, zi + 5],
                                   0.5 * (du * rdy + dv * rdx))
                sm = sm + du * du + dv * dv + dp * dp + dq * dq
            return sm

        smooth = lax.fori_loop(0, GPC, group_body, smooth)
        dss = [pltpu.async_copy(chan.at[pl.ds(g * G, G)],
                                acc_sh.at[rowi.at[g]], ssem, add=True)
               for g in range(GPC)]
        for d in dss:
            d.wait()
        return smooth

    smooth = lax.fori_loop(0, NCHUNK, chunk_body,
                           jnp.zeros((L,), jnp.float32))
    smbuf[...] = smooth
    pltpu.sync_copy(smbuf, sm_out.at[cid, sid])
    plsc.subcore_barrier()
    pltpu.sync_copy(acc_sh.at[pl.ds(sid * RPS, RPS)],
                    acc_out.at[cid, pl.ds(sid * RPS, RPS)])


def _node_body(pred_t, tgt_t, wallp, acc_in, part_out,
               ub, vb, pb, qb, tub, tvb, tpb, tqb, wb, a0, a1, pout):
    cid = lax.axis_index("c")
    sid = lax.axis_index("s")
    wid = cid * NS + sid
    base = wid * RPW
    for f, b in enumerate((ub, vb, pb, qb)):
        pltpu.sync_copy(pred_t.at[f, pl.ds(base, RPW)], b)
    for f, b in enumerate((tub, tvb, tpb, tqb)):
        pltpu.sync_copy(tgt_t.at[f, pl.ds(base, RPW)], b)
    pltpu.sync_copy(wallp.at[pl.ds(base, RPW)], wb)
    pltpu.sync_copy(acc_in.at[0, pl.ds(base, RPW)], a0)
    pltpu.sync_copy(acc_in.at[1, pl.ds(base, RPW)], a1)

    iota = lax.iota(jnp.int32, L)
    zi = jnp.zeros((L,), jnp.int32)
    zf = jnp.zeros((L,), jnp.float32)

    def step(i, carry):
        mse, dv2, mom, tpd, tds, bcs, wls, wsm = carry
        s = i * L
        e16 = zi + s + iota

        def ga(ref, f):
            return plsc.load_gather(ref, [e16, zi + f])

        u = ub[pl.ds(s, L)]
        v = vb[pl.ds(s, L)]
        p = pb[pl.ds(s, L)]
        q = qb[pl.ds(s, L)]
        tu = tub[pl.ds(s, L)]
        tv = tvb[pl.ds(s, L)]
        tp = tpb[pl.ds(s, L)]
        tq = tqb[pl.ds(s, L)]
        w = wb[pl.ds(s, L)]
        cnt = jnp.maximum(ga(a0, 6) + ga(a1, 6), 1.0)
        inv = 1.0 / cnt
        dvg = (ga(a0, 0) + ga(a1, 0)) * inv
        px = (ga(a0, 1) + ga(a1, 1)) * inv
        py = (ga(a0, 2) + ga(a1, 2)) * inv
        lu = (ga(a0, 3) + ga(a1, 3)) * inv
        lv = (ga(a0, 4) + ga(a1, 4)) * inv
        st = (ga(a0, 5) + ga(a1, 5)) * inv
        rq = jnp.maximum(q, 0.0)
        nue = NU_MOL + rq
        rx = px - nue * lu
        ry = py - nue * lv
        nq = jnp.minimum(q, 0.0)
        buv = w * (u * u + v * v)
        eu = u - tu
        ev = v - tv
        ep = p - tp
        eq = q - tq
        rqst = rq * st
        mse = mse + eu * eu + ev * ev + ep * ep + eq * eq
        dv2 = dv2 + dvg * dvg
        mom = mom + rx * rx + ry * ry
        tpd = tpd + nq * nq
        tds = tds + rqst * rqst
        bcs = bcs + buv
        wls = wls + buv + w * q * q
        wsm = wsm + w
        return (mse, dv2, mom, tpd, tds, bcs, wls, wsm)

    res = lax.fori_loop(0, RPW // L, step, (zf,) * 8)
    for k, vec in enumerate(res):
        pout[k] = vec
    pltpu.sync_copy(pout, part_out.at[cid, sid])


_MESH = plsc.VectorSubcoreMesh(core_axis_name="c", subcore_axis_name="s")
_PARAMS = pltpu.CompilerParams(use_tc_tiling_on_sc=False,
                               needs_layout_passes=False)

_edge_kernel = pl.kernel(
    _edge_body,
    out_type=[
        jax.ShapeDtypeStruct((NC, NPAD, 8), jnp.float32),
        jax.ShapeDtypeStruct((NC, NS, L), jnp.float32),
    ],
    mesh=_MESH,
    compiler_params=_PARAMS,
    scratch_types=[
        pltpu.VMEM((GPC, G), jnp.int32),
        pltpu.VMEM((GPC, G), jnp.int32),
        pltpu.VMEM((GPC, G, 2), jnp.float32),
        pltpu.VMEM((K, 8), jnp.float32),
        pltpu.VMEM((K, 8), jnp.float32),
        pltpu.VMEM((K, 8), jnp.float32),
        pltpu.VMEM((L,), jnp.float32),
        pltpu.VMEM_SHARED((NPAD, 8), jnp.float32),
        pltpu.SemaphoreType.DMA,
        pltpu.SemaphoreType.DMA,
    ],
)

_node_kernel = pl.kernel(
    _node_body,
    out_type=[jax.ShapeDtypeStruct((NC, NS, 8, L), jnp.float32)],
    mesh=_MESH,
    compiler_params=_PARAMS,
    scratch_types=(
        [pltpu.VMEM((RPW,), jnp.float32) for _ in range(9)]
        + [pltpu.VMEM((RPW, 8), jnp.float32) for _ in range(2)]
        + [pltpu.VMEM((8, L), jnp.float32)]
    ),
)


def kernel(predictions, targets, edge_index, edge_attr_dxdy, wall_mask, step):
    step_f = jnp.asarray(step, dtype=jnp.float32)
    alpha = jnp.clip(step_f / 1000.0, 0.0, 1.0)
    cont_w = (1.0 - alpha) * 0.1 + alpha * 0.1
    mom_w = (1.0 - alpha) * 0.1 + alpha * 0.1

    pred_pad = jnp.pad(predictions, ((0, NPAD - N), (0, 4)))
    tgt_pad = jnp.pad(targets, ((0, NPAD - N), (0, 0)))
    wall_f = jnp.pad(wall_mask.astype(jnp.float32), (0, NPAD - N))
    pred_t = pred_pad[:, :4].T.copy()
    tgt_t = tgt_pad.T.copy()
    ei3 = edge_index.reshape(2, EG, G)
    attr3 = edge_attr_dxdy.reshape(EG, G, 2)
    zrows = jnp.zeros((RPS, 8), jnp.float32)
    cconst = jnp.tile(
        jnp.array([0, 0, 0, 0, 0, 0, 1, 0], jnp.float32), (K, 1))

    acc, sm = _edge_kernel(ei3, attr3, pred_pad, zrows, cconst)
    (parts,) = _node_kernel(pred_t, tgt_t, wall_f, acc)

    ps = parts.sum(axis=(0, 1, 3))
    mse_s, dv2_s, mom_s, tpd_s, tds_s, bc_s, wl_s, wsum = (
        ps[0], ps[1], ps[2], ps[3], ps[4], ps[5], ps[6], ps[7])
    denom = jnp.maximum(wsum, 1.0)
    smooth = sm.sum() / (4.0 * E)
    total = (1.0 * mse_s / (4.0 * N)
             + cont_w * dv2_s / N
             + mom_w * mom_s / N
             + 0.05 * tpd_s / N
             + 0.05 * tds_s / N
             + 0.05 * bc_s / denom
             + 0.01 * smooth
             + 0.02 * wl_s / denom)
    return total
